# baseline (speedup 1.0000x reference)
import functools

import jax
import jax.numpy as jnp
from jax import lax
from jax.experimental import pallas as pl
from jax.experimental.pallas import tpu as pltpu
from jax.experimental.pallas import tpu_sc as plsc

_FEAT = 128
_PIDS = 100000
_CQ = 5000
_SCALAR = 30.0
_B = 128

_TILE = 5000
_T_LUT = _PIDS // _TILE
_NS = 4
_SPAN = _T_LUT // _NS
_GRID = _SPAN + 1
_IGNORE = 5554
_LN2 = 0.6931471805599453

_NW = 8
_RPW = _B // _NW


def _lane_shuffle(v, perm):
    return lax.gather(
        v, perm[:, None],
        lax.GatherDimensionNumbers(
            offset_dims=(), collapsed_slice_dims=(0,), start_index_map=(0,)),
        (1,), mode=lax.GatherScatterMode.PROMISE_IN_BOUNDS)


@functools.partial(
    pl.kernel,
    out_type=jax.ShapeDtypeStruct((_B,), jnp.float32),
    mesh=plsc.VectorSubcoreMesh(core_axis_name="c", subcore_axis_name="s"),
    scratch_types=[
        pltpu.VMEM((_RPW,), jnp.int32),
        pltpu.VMEM((_RPW, _FEAT), jnp.float32),
        pltpu.VMEM((_RPW, _FEAT), jnp.float32),
        pltpu.VMEM((_RPW,), jnp.float32),
        pltpu.VMEM((_RPW,), jnp.float32),
        pltpu.SemaphoreType.DMA,
    ],
)
def _sc_label_scores(x_hbm, lbl_hbm, c_hbm, lut_hbm, out_hbm,
                     idx_v, rows_v, x_v, c_v, t_v, sem):
    wid = lax.axis_index("s") * 2 + lax.axis_index("c")

    @pl.when(wid < _NW)
    def _():
        base = wid * _RPW
        pltpu.sync_copy(lbl_hbm.at[pl.ds(base, _RPW)], idx_v)
        pltpu.sync_copy(x_hbm.at[pl.ds(base, _RPW), :], x_v)
        pltpu.async_copy(lut_hbm.at[idx_v], rows_v, sem).wait()
        pltpu.async_copy(c_hbm.at[idx_v], c_v, sem).wait()
        lane = lax.iota(jnp.int32, 16)
        t = jnp.zeros((16,), jnp.float32)
        for r in range(_RPW):
            acc = jnp.zeros((16,), jnp.float32)
            for k in range(_FEAT // 16):
                acc = acc + (x_v[r, pl.ds(k * 16, 16)]
                             * rows_v[r, pl.ds(k * 16, 16)])
            for sh in (8, 4, 2, 1):
                acc = acc + _lane_shuffle(acc, lane ^ sh)
            t = jnp.where(lane == r, acc, t)
        t_v[...] = t * c_v[...]
        pltpu.sync_copy(t_v, out_hbm.at[pl.ds(base, _RPW)])


def _oim_body(m2_ref, x_ref, lbl_ref, c0_ref, c1_ref, c2_ref, c3_ref,
              w0_ref, w1_ref, w2_ref, w3_ref, cq_ref, out_ref, s_s):
    i = pl.program_id(0)

    @pl.when(i == 0)
    def _init():
        s_s[...] = jnp.zeros((_B, 1), jnp.float32)

    x = x_ref[...]
    m2 = m2_ref[0]

    def _accumulate(w, c):
        s2 = jax.lax.dot_general(
            x, w.astype(jnp.bfloat16), (((1,), (1,)), ((), ())),
            preferred_element_type=jnp.float32)
        s2 = s2 * c[None, :]
        p = jnp.exp2(s2 - m2)
        s_s[...] += jnp.sum(p, axis=1, keepdims=True)

    @pl.when(i < _SPAN)
    def _lut_phase():
        _accumulate(w0_ref[...], c0_ref[0, 0, :])
        _accumulate(w1_ref[...], c1_ref[0, 0, :])
        _accumulate(w2_ref[...], c2_ref[0, 0, :])
        _accumulate(w3_ref[...], c3_ref[0, 0, :])

    @pl.when(i == _SPAN)
    def _cq_phase():
        _accumulate(cq_ref[...], c0_ref[0, 0, :])

    @pl.when(i == _GRID - 1)
    def _finish():
        lse = m2 * _LN2 + jnp.log(s_s[...])
        valid = (lbl_ref[...] != _IGNORE).astype(jnp.float32)
        denom = jnp.maximum(jnp.sum(valid), 1.0)
        a = jnp.sum(lse * valid)
        out_ref[...] = jnp.concatenate(
            [a.reshape(1, 1), denom.reshape(1, 1)], axis=1)


def _c_spec(k):
    return pl.BlockSpec(
        (1, 1, _TILE),
        lambda i, k=k: (jnp.where(i < _SPAN, i + k * _SPAN, _T_LUT), 0, 0))


def _w_spec(k):
    return pl.BlockSpec(
        (_TILE, _FEAT),
        lambda i, k=k: (jnp.minimum(i, _SPAN - 1) + k * _SPAN, 0))


def kernel(inputs, roi_label, roi_ious, lut, cq, reliability):
    del roi_ious
    lbl1 = roi_label.reshape(_B).astype(jnp.int32) - 1
    lbl = lbl1.reshape(_B, 1)
    c = reliability * (_SCALAR * 1.4426950408889634)
    m2 = jnp.max(jnp.abs(c)).reshape(1)
    c3 = c.reshape(_T_LUT + 1, 1, _TILE)
    t2 = _sc_label_scores(inputs, lbl1, c, lut)
    xb = inputs.astype(jnp.bfloat16)
    out = pl.pallas_call(
        _oim_body,
        grid=(_GRID,),
        in_specs=[
            pl.BlockSpec(memory_space=pltpu.SMEM),
            pl.BlockSpec((_B, _FEAT), lambda i: (0, 0)),
            pl.BlockSpec((_B, 1), lambda i: (0, 0)),
            _c_spec(0), _c_spec(1), _c_spec(2), _c_spec(3),
            _w_spec(0), _w_spec(1), _w_spec(2), _w_spec(3),
            pl.BlockSpec((_CQ, _FEAT), lambda i: (0, 0)),
        ],
        out_specs=pl.BlockSpec((1, 2), lambda i: (0, 0)),
        out_shape=jax.ShapeDtypeStruct((1, 2), jnp.float32),
        scratch_shapes=[
            pltpu.VMEM((_B, 1), jnp.float32),
        ],
    )(m2, xb, lbl, c3, c3, c3, c3, lut, lut, lut, lut, cq)
    valid = (lbl1 != _IGNORE).astype(jnp.float32)
    return (out[0, 0] - jnp.sum(t2 * valid) * _LN2) / out[0, 1]


# speedup vs baseline: 1.5484x; 1.5484x over previous
import jax
import jax.numpy as jnp
from jax.experimental import pallas as pl
from jax.experimental.pallas import tpu as pltpu

_FEAT = 128
_PIDS = 100000
_CQ = 5000
_SCALAR = 30.0
_B = 128

_TILE = 5000
_T_LUT = _PIDS // _TILE
_NS = 4
_SPAN = _T_LUT // _NS
_GRID = _SPAN
_IGNORE = 5554
_LN2 = 0.6931471805599453


def _oim_body(m2_ref, x_ref, lbl_ref, c0_ref, c1_ref, c2_ref, c3_ref, cc_ref,
              w0_ref, w1_ref, w2_ref, w3_ref, cq_ref, out_ref, s_s, t_s):
    i = pl.program_id(0)

    @pl.when(i == 0)
    def _init():
        s_s[...] = jnp.zeros((_B, 1), jnp.float32)
        t_s[...] = jnp.zeros((_B, 1), jnp.float32)

    x = x_ref[...]
    m2 = m2_ref[0]

    def _accumulate(w, c, base, with_target):
        s2 = jax.lax.dot_general(
            x, w.astype(jnp.bfloat16), (((1,), (1,)), ((), ())),
            preferred_element_type=jnp.float32)
        s2 = s2 * c[None, :]
        p = jnp.exp2(s2 - m2)
        s_s[...] += jnp.sum(p, axis=1, keepdims=True)
        if with_target:
            col = lbl_ref[...] - base
            iota = jax.lax.broadcasted_iota(jnp.int32, (_B, _TILE), 1)
            hit = jnp.where(iota == col, s2, 0.0)
            t_s[...] += jnp.sum(hit, axis=1, keepdims=True)

    _accumulate(w0_ref[...], c0_ref[0, 0, :], i * _TILE, True)
    _accumulate(w1_ref[...], c1_ref[0, 0, :], (i + _SPAN) * _TILE, True)
    _accumulate(w2_ref[...], c2_ref[0, 0, :], (i + 2 * _SPAN) * _TILE, True)
    _accumulate(w3_ref[...], c3_ref[0, 0, :], (i + 3 * _SPAN) * _TILE, True)

    @pl.when(i == 0)
    def _cq_once():
        _accumulate(cq_ref[...], cc_ref[0, 0, :], _PIDS, False)

    @pl.when(i == _GRID - 1)
    def _finish():
        lse = m2 * _LN2 + jnp.log(s_s[...])
        nll = lse - t_s[...] * _LN2
        valid = (lbl_ref[...] != _IGNORE).astype(jnp.float32)
        denom = jnp.maximum(jnp.sum(valid), 1.0)
        out_ref[...] = (jnp.sum(nll * valid) / denom).reshape(1, 1)


def _c_spec(k):
    return pl.BlockSpec((1, 1, _TILE), lambda i, k=k: (i + k * _SPAN, 0, 0))


def _w_spec(k):
    return pl.BlockSpec((_TILE, _FEAT), lambda i, k=k: (i + k * _SPAN, 0))


def kernel(inputs, roi_label, roi_ious, lut, cq, reliability):
    del roi_ious
    lbl = roi_label.reshape(_B, 1).astype(jnp.int32) - 1
    xb = inputs.astype(jnp.bfloat16)
    c = reliability * (_SCALAR * 1.4426950408889634)
    m2 = jnp.max(jnp.abs(c)).reshape(1)
    c3 = c.reshape(_T_LUT + 1, 1, _TILE)
    out = pl.pallas_call(
        _oim_body,
        grid=(_GRID,),
        in_specs=[
            pl.BlockSpec(memory_space=pltpu.SMEM),
            pl.BlockSpec((_B, _FEAT), lambda i: (0, 0)),
            pl.BlockSpec((_B, 1), lambda i: (0, 0)),
            _c_spec(0), _c_spec(1), _c_spec(2), _c_spec(3),
            pl.BlockSpec((1, 1, _TILE), lambda i: (_T_LUT, 0, 0)),
            _w_spec(0), _w_spec(1), _w_spec(2), _w_spec(3),
            pl.BlockSpec((_CQ, _FEAT), lambda i: (0, 0)),
        ],
        out_specs=pl.BlockSpec((1, 1), lambda i: (0, 0)),
        out_shape=jax.ShapeDtypeStruct((1, 1), jnp.float32),
        scratch_shapes=[
            pltpu.VMEM((_B, 1), jnp.float32),
            pltpu.VMEM((_B, 1), jnp.float32),
        ],
    )(m2, xb, lbl, c3, c3, c3, c3, c3, lut, lut, lut, lut, cq)
    return out[0, 0]


# speedup vs baseline: 1.5727x; 1.0157x over previous
import jax
import jax.numpy as jnp
from jax.experimental import pallas as pl
from jax.experimental.pallas import tpu as pltpu

_FEAT = 128
_PIDS = 100000
_CQ = 5000
_SCALAR = 30.0
_B = 128

_TILE = 5000
_T_LUT = _PIDS // _TILE
_NS = 5
_SPAN = _T_LUT // _NS
_GRID = _SPAN
_IGNORE = 5554
_LN2 = 0.6931471805599453


def _oim_body(m2_ref, x_ref, lbl_ref, c0_ref, c1_ref, c2_ref, c3_ref, c4_ref,
              cc_ref, w0_ref, w1_ref, w2_ref, w3_ref, w4_ref, cq_ref, out_ref,
              s_s, t_s):
    i = pl.program_id(0)

    @pl.when(i == 0)
    def _init():
        s_s[...] = jnp.zeros((_B, 1), jnp.float32)
        t_s[...] = jnp.zeros((_B, 1), jnp.float32)

    x = x_ref[...]
    m2 = m2_ref[0]

    def _accumulate(w, c, base, with_target):
        s2 = jax.lax.dot_general(
            x, w.astype(jnp.bfloat16), (((1,), (1,)), ((), ())),
            preferred_element_type=jnp.float32)
        s2 = s2 * c[None, :]
        p = jnp.exp2(s2 - m2)
        s_s[...] += jnp.sum(p, axis=1, keepdims=True)
        if with_target:
            col = lbl_ref[...] - base
            iota = jax.lax.broadcasted_iota(jnp.int32, (_B, _TILE), 1)
            hit = jnp.where(iota == col, s2, 0.0)
            t_s[...] += jnp.sum(hit, axis=1, keepdims=True)

    _accumulate(w0_ref[...], c0_ref[0, 0, :], i * _TILE, True)
    _accumulate(w1_ref[...], c1_ref[0, 0, :], (i + _SPAN) * _TILE, True)
    _accumulate(w2_ref[...], c2_ref[0, 0, :], (i + 2 * _SPAN) * _TILE, True)
    _accumulate(w3_ref[...], c3_ref[0, 0, :], (i + 3 * _SPAN) * _TILE, True)
    _accumulate(w4_ref[...], c4_ref[0, 0, :], (i + 4 * _SPAN) * _TILE, True)

    @pl.when(i == 0)
    def _cq_once():
        _accumulate(cq_ref[...], cc_ref[0, 0, :], _PIDS, False)

    @pl.when(i == _GRID - 1)
    def _finish():
        lse = m2 * _LN2 + jnp.log(s_s[...])
        nll = lse - t_s[...] * _LN2
        valid = (lbl_ref[...] != _IGNORE).astype(jnp.float32)
        denom = jnp.maximum(jnp.sum(valid), 1.0)
        out_ref[...] = (jnp.sum(nll * valid) / denom).reshape(1, 1)


def _c_spec(k):
    return pl.BlockSpec((1, 1, _TILE), lambda i, k=k: (i + k * _SPAN, 0, 0))


def _w_spec(k):
    return pl.BlockSpec((_TILE, _FEAT), lambda i, k=k: (i + k * _SPAN, 0))


def kernel(inputs, roi_label, roi_ious, lut, cq, reliability):
    del roi_ious
    lbl = roi_label.reshape(_B, 1).astype(jnp.int32) - 1
    xb = inputs.astype(jnp.bfloat16)
    c = reliability * (_SCALAR * 1.4426950408889634)
    m2 = jnp.max(jnp.abs(c)).reshape(1)
    c3 = c.reshape(_T_LUT + 1, 1, _TILE)
    out = pl.pallas_call(
        _oim_body,
        grid=(_GRID,),
        in_specs=[
            pl.BlockSpec(memory_space=pltpu.SMEM),
            pl.BlockSpec((_B, _FEAT), lambda i: (0, 0)),
            pl.BlockSpec((_B, 1), lambda i: (0, 0)),
            _c_spec(0), _c_spec(1), _c_spec(2), _c_spec(3), _c_spec(4),
            pl.BlockSpec((1, 1, _TILE), lambda i: (_T_LUT, 0, 0)),
            _w_spec(0), _w_spec(1), _w_spec(2), _w_spec(3), _w_spec(4),
            pl.BlockSpec((_CQ, _FEAT), lambda i: (0, 0)),
        ],
        out_specs=pl.BlockSpec((1, 1), lambda i: (0, 0)),
        out_shape=jax.ShapeDtypeStruct((1, 1), jnp.float32),
        scratch_shapes=[
            pltpu.VMEM((_B, 1), jnp.float32),
            pltpu.VMEM((_B, 1), jnp.float32),
        ],
    )(m2, xb, lbl, c3, c3, c3, c3, c3, c3, lut, lut, lut, lut, lut, cq)
    return out[0, 0]


# speedup vs baseline: 1.5825x; 1.0062x over previous
import jax
import jax.numpy as jnp
from jax.experimental import pallas as pl
from jax.experimental.pallas import tpu as pltpu

_FEAT = 128
_PIDS = 100000
_CQ = 5000
_SCALAR = 30.0
_B = 128

_TILE = 2000
_T_LUT = _PIDS // _TILE
_NS = 10
_SPAN = _T_LUT // _NS
_GRID = _SPAN
_IGNORE = 5554
_LN2 = 0.6931471805599453


def _oim_body(*refs):
    m2_ref, x_ref, lbl_ref = refs[0:3]
    c_refs = refs[3:3 + _NS]
    cc_ref = refs[3 + _NS]
    w_refs = refs[4 + _NS:4 + 2 * _NS]
    cq_ref = refs[4 + 2 * _NS]
    out_ref = refs[5 + 2 * _NS]
    s_s, t_s = refs[6 + 2 * _NS:8 + 2 * _NS]
    i = pl.program_id(0)

    @pl.when(i == 0)
    def _init():
        s_s[...] = jnp.zeros((_B, 1), jnp.float32)
        t_s[...] = jnp.zeros((_B, 1), jnp.float32)

    x = x_ref[...]
    m2 = m2_ref[0]

    def _accumulate(w, c, base, tile, with_target):
        s2 = jax.lax.dot_general(
            x, w.astype(jnp.bfloat16), (((1,), (1,)), ((), ())),
            preferred_element_type=jnp.float32)
        s2 = s2 * c[None, :]
        p = jnp.exp2(s2 - m2)
        s_s[...] += jnp.sum(p, axis=1, keepdims=True)
        if with_target:
            col = lbl_ref[...] - base
            iota = jax.lax.broadcasted_iota(jnp.int32, (_B, tile), 1)
            hit = jnp.where(iota == col, s2, 0.0)
            t_s[...] += jnp.sum(hit, axis=1, keepdims=True)

    for k in range(_NS):
        _accumulate(w_refs[k][...], c_refs[k][0, 0, :],
                    (i + k * _SPAN) * _TILE, _TILE, True)

    @pl.when(i == 0)
    def _cq_once():
        _accumulate(cq_ref[...], cc_ref[0, 0, :], _PIDS, _CQ, False)

    @pl.when(i == _GRID - 1)
    def _finish():
        lse = m2 * _LN2 + jnp.log(s_s[...])
        nll = lse - t_s[...] * _LN2
        valid = (lbl_ref[...] != _IGNORE).astype(jnp.float32)
        denom = jnp.maximum(jnp.sum(valid), 1.0)
        out_ref[...] = (jnp.sum(nll * valid) / denom).reshape(1, 1)


def _c_spec(k):
    return pl.BlockSpec((1, 1, _TILE), lambda i, k=k: (i + k * _SPAN, 0, 0))


def _w_spec(k):
    return pl.BlockSpec((_TILE, _FEAT), lambda i, k=k: (i + k * _SPAN, 0))


def kernel(inputs, roi_label, roi_ious, lut, cq, reliability):
    del roi_ious
    lbl = roi_label.reshape(_B, 1).astype(jnp.int32) - 1
    xb = inputs.astype(jnp.bfloat16)
    c = reliability * (_SCALAR * 1.4426950408889634)
    m2 = jnp.max(jnp.abs(c)).reshape(1)
    c_lut = c[:_PIDS].reshape(_T_LUT, 1, _TILE)
    c_cq = c[_PIDS:].reshape(1, 1, _CQ)
    out = pl.pallas_call(
        _oim_body,
        grid=(_GRID,),
        in_specs=[
            pl.BlockSpec(memory_space=pltpu.SMEM),
            pl.BlockSpec((_B, _FEAT), lambda i: (0, 0)),
            pl.BlockSpec((_B, 1), lambda i: (0, 0)),
            *[_c_spec(k) for k in range(_NS)],
            pl.BlockSpec((1, 1, _CQ), lambda i: (0, 0, 0)),
            *[_w_spec(k) for k in range(_NS)],
            pl.BlockSpec((_CQ, _FEAT), lambda i: (0, 0)),
        ],
        out_specs=pl.BlockSpec((1, 1), lambda i: (0, 0)),
        out_shape=jax.ShapeDtypeStruct((1, 1), jnp.float32),
        scratch_shapes=[
            pltpu.VMEM((_B, 1), jnp.float32),
            pltpu.VMEM((_B, 1), jnp.float32),
        ],
    )(m2, xb, lbl, *([c_lut] * _NS), c_cq, *([lut] * _NS), cq)
    return out[0, 0]
